# baseline (speedup 1.0000x reference)
import jax
import jax.numpy as jnp
from jax.experimental import pallas as pl

_B, _N, _F = 2048, 64, 64
_S, _SEGF, _OUTD, _H, _K = 4, 128, 128, 4, 4
_DH = _F // _H
_BLK = 128
_R = _BLK * _N
_STEPS = _B // _BLK
_TOT = float(_B * _N)

_INTERPRET = False


def _batched_dot(a, b, ca, cb):
    return jax.lax.dot_general(a, b, (((ca,), (cb,)), ((0,), (0,))))


def _compute_assign_t(x, sq, wq, bq, wk, bk):
    q = jax.lax.dot(sq, wq) + bq
    kk = jax.lax.dot(x, wk) + bk
    qt = jnp.transpose(q)
    qtile = jnp.concatenate([qt] * _H, axis=1)
    rows = jax.lax.broadcasted_iota(jnp.int32, (_F, _H * _S), 0) // _DH
    cols = jax.lax.broadcasted_iota(jnp.int32, (_F, _H * _S), 1) // _S
    qblk = jnp.where(rows == cols, qtile, 0.0)
    sc = jax.lax.dot(kk, qblk) * (1.0 / (_DH ** 0.5))
    sc3 = jnp.transpose(sc.reshape(_BLK, _N, _H * _S), (0, 2, 1))
    m = jnp.max(sc3, axis=2, keepdims=True)
    e = jnp.exp(sc3 - m)
    p = e / jnp.sum(e, axis=2, keepdims=True)
    attn = (p[:, 0 * _S:1 * _S] + p[:, 1 * _S:2 * _S] +
            p[:, 2 * _S:3 * _S] + p[:, 3 * _S:4 * _S]) * (1.0 / _H)
    m2 = jnp.max(attn, axis=1, keepdims=True)
    e2 = jnp.exp(attn - m2)
    return e2 / jnp.sum(e2, axis=1, keepdims=True)


def _knn_adj(px, py):
    dx = px[:, :, None] - px[:, None, :]
    dy = py[:, :, None] - py[:, None, :]
    d = jnp.sqrt(dx * dx + dy * dy + 1e-12)
    flane = jax.lax.broadcasted_iota(
        jnp.int32, (_BLK, _N, _N), 2).astype(jnp.float32)
    dcur = d
    sel0 = None
    for t in range(_K + 1):
        m = jnp.min(dcur, axis=2, keepdims=True)
        idxb = jnp.min(jnp.where(dcur == m, flane, float(_N)),
                       axis=2, keepdims=True)
        sel = flane == idxb
        if t == 0:
            sel0 = sel
        dcur = jnp.where(sel, jnp.inf, dcur)
    adj = jnp.where(sel0, 0.0,
                    jnp.where(dcur == jnp.inf, 1.0, 0.0))
    ones3 = jnp.ones((_BLK, 1, _N), jnp.float32)
    deg = _batched_dot(ones3, adj, 2, 1).reshape(_BLK, _N)
    deg = jnp.maximum(deg, 1.0)
    return adj, 1.0 / deg


def _message_from_adj(out, adj3, rdeg):
    o3 = out.reshape(_BLK, _N, _F)
    msgs = _batched_dot(adj3, o3, 1, 1)
    z3 = o3 + msgs * rdeg[:, :, None]
    return z3.reshape(_R, _F)


def _accum_stats(st_ref, z):
    @pl.when(pl.program_id(0) == 0)
    def _():
        st_ref[...] = jnp.zeros((8, 128), jnp.float32)
    ones = jnp.ones((1, _R), jnp.float32)
    s = jax.lax.dot(ones, z)
    q = jax.lax.dot(ones, z * z)
    upd = jnp.concatenate([s, q], axis=1)
    st_ref[0:1, :] = st_ref[0:1, :] + upd


def _norm_from_stats(st_ref, z, g, beta):
    st = st_ref[0:1, :]
    mu = st[:, 0:_F] * (1.0 / _TOT)
    var = st[:, _F:128] * (1.0 / _TOT) - mu * mu
    rstd = jax.lax.rsqrt(var + 1e-5)
    a = g * rstd
    b = beta - mu * a
    return jnp.maximum(a * z + b, 0.0)


def _seg_body(x_ref, pos_ref, sq_ref, wq_ref, bq_ref, wk_ref,
              bk_ref, a2sw_ref, a2sb_ref, seg_ref, at_ref, adj_ref,
              rdeg_ref):
    x = x_ref[...]
    at = _compute_assign_t(x, sq_ref[...], wq_ref[...], bq_ref[...],
                           wk_ref[...], bk_ref[...])
    at_ref[...] = at.reshape(_BLK * _S, _N)
    proj = jax.lax.dot(x, a2sw_ref[...]) + a2sb_ref[...]
    proj3 = proj.reshape(_BLK, _N, _SEGF)
    seg = _batched_dot(at, proj3, 2, 1)
    seg_ref[...] = seg.reshape(_BLK * _S, _SEGF)
    pos = pos_ref[...]
    adj, rdeg = _knn_adj(pos[:, :, 0], pos[:, :, 1])
    adj_ref[...] = adj.reshape(_R, _N)
    rdeg_ref[...] = rdeg


def _coarse_body(sf_ref, w0, b0, g0, be0, w1, b1, g1, be1, out_ref):
    sf = sf_ref[...]
    for (w, b, g, be) in ((w0, b0, g0, be0), (w1, b1, g1, be1)):
        out = jax.lax.dot(sf, w[...]) + b[...]
        o3 = out.reshape(_B, _S, _SEGF)
        tot = jnp.sum(o3, axis=1, keepdims=True)
        z = (o3 + (tot - o3) * (1.0 / (_S - 1))).reshape(_B * _S, _SEGF)
        mu = jnp.mean(z, axis=0, keepdims=True)
        var = jnp.mean((z - mu) ** 2, axis=0, keepdims=True)
        sf = jnp.maximum(
            g[...] * (z - mu) * jax.lax.rsqrt(var + 1e-5) + be[...], 0.0)
    out_ref[...] = sf


def _fine0_body(x_ref, adj_ref, rdeg_ref, at_ref, seg2_ref, s2aw_ref,
                s2ab_ref, w_ref, b_ref, z_ref, st_ref):
    x = x_ref[...]
    at = at_ref[...].reshape(_BLK, _S, _N)
    seg2 = seg2_ref[...].reshape(_BLK, _S, _SEGF)
    segctx = _batched_dot(at, seg2, 1, 1)
    segctx = segctx.reshape(_R, _SEGF)
    enh = x + jax.lax.dot(segctx, s2aw_ref[...]) + s2ab_ref[...]
    out = jax.lax.dot(enh, w_ref[...]) + b_ref[...]
    z = _message_from_adj(out, adj_ref[...].reshape(_BLK, _N, _N),
                          rdeg_ref[...])
    z_ref[...] = z
    _accum_stats(st_ref, z)


def _fine_body(z_in_ref, stin_ref, adj_ref, rdeg_ref,
               g_ref, be_ref, w_ref, b_ref, z_ref, st_ref):
    y = _norm_from_stats(stin_ref, z_in_ref[...], g_ref[...], be_ref[...])
    out = jax.lax.dot(y, w_ref[...]) + b_ref[...]
    z = _message_from_adj(out, adj_ref[...].reshape(_BLK, _N, _N),
                          rdeg_ref[...])
    z_ref[...] = z
    _accum_stats(st_ref, z)


def _pool_body(z_in_ref, stin_ref, pres_ref, g_ref, be_ref,
               ow_ref, ob_ref, out_ref):
    y = _norm_from_stats(stin_ref, z_in_ref[...], g_ref[...], be_ref[...])
    y3 = y.reshape(_BLK, _N, _F)
    pw = pres_ref[...]
    num = _batched_dot(pw.reshape(_BLK, 1, _N), y3, 2, 1)
    num = num.reshape(_BLK, _F)
    den = jnp.maximum(jnp.sum(pw, axis=1, keepdims=True), 1e-8)
    gf = num / den
    out_ref[...] = jax.lax.dot(gf, ow_ref[...]) + ob_ref[...]


@jax.jit
def kernel(axle_features, axle_positions, presence_scores, seg_queries,
           mha_wq, mha_bq, mha_wk, mha_bk, a2s_w, a2s_b, s2a_w, s2a_b,
           coarse_w, coarse_b, coarse_g, coarse_beta,
           fine_w, fine_b, fine_g, fine_beta, out_w, out_b):
    f32 = jnp.float32
    x2 = axle_features.reshape(_B * _N, _F)
    row = lambda a: a.reshape(1, -1)

    grid = (_STEPS,)
    xspec = pl.BlockSpec((_R, _F), lambda i: (i, 0))
    pspec = pl.BlockSpec((_BLK, _N), lambda i: (i, 0))
    posspec = pl.BlockSpec((_BLK, _N, 2), lambda i: (i, 0, 0))
    segspec = pl.BlockSpec((_BLK * _S, _SEGF), lambda i: (i, 0))
    atspec = pl.BlockSpec((_BLK * _S, _N), lambda i: (i, 0))
    stspec = pl.BlockSpec((8, 128), lambda i: (0, 0))
    w = lambda: pl.BlockSpec(None, lambda i: (0, 0))

    seg, at, adj, rdeg = pl.pallas_call(
        _seg_body, grid=grid,
        in_specs=[xspec, posspec] + [w()] * 7,
        out_specs=[segspec, atspec, xspec, pspec],
        out_shape=[jax.ShapeDtypeStruct((_B * _S, _SEGF), f32),
                   jax.ShapeDtypeStruct((_B * _S, _N), f32),
                   jax.ShapeDtypeStruct((_B * _N, _N), f32),
                   jax.ShapeDtypeStruct((_B, _N), f32)],
        interpret=_INTERPRET,
    )(x2, axle_positions, seg_queries, mha_wq, row(mha_bq), mha_wk,
      row(mha_bk), a2s_w, row(a2s_b))

    seg2 = pl.pallas_call(
        _coarse_body,
        in_specs=[pl.BlockSpec((_B * _S, _SEGF), lambda: (0, 0))] +
                 [pl.BlockSpec(None, lambda: (0, 0))] * 8,
        out_specs=pl.BlockSpec((_B * _S, _SEGF), lambda: (0, 0)),
        out_shape=jax.ShapeDtypeStruct((_B * _S, _SEGF), f32),
        interpret=_INTERPRET,
    )(seg, coarse_w[0], row(coarse_b[0]), row(coarse_g[0]),
      row(coarse_beta[0]), coarse_w[1], row(coarse_b[1]),
      row(coarse_g[1]), row(coarse_beta[1]))

    zshape = jax.ShapeDtypeStruct((_B * _N, _F), f32)
    stshape = jax.ShapeDtypeStruct((8, 128), f32)

    z1, st1 = pl.pallas_call(
        _fine0_body, grid=grid,
        in_specs=[xspec, xspec, pspec, atspec, segspec, w(), w(), w(), w()],
        out_specs=[xspec, stspec],
        out_shape=[zshape, stshape],
        interpret=_INTERPRET,
    )(x2, adj, rdeg, at, seg2, s2a_w, row(s2a_b), fine_w[0],
      row(fine_b[0]))

    zc, stc = z1, st1
    for i in (1, 2):
        zc, stc = pl.pallas_call(
            _fine_body, grid=grid,
            in_specs=[xspec, stspec, xspec, pspec, w(), w(), w(), w()],
            out_specs=[xspec, stspec],
            out_shape=[zshape, stshape],
            interpret=_INTERPRET,
        )(zc, stc, adj, rdeg, row(fine_g[i - 1]), row(fine_beta[i - 1]),
          fine_w[i], row(fine_b[i]))

    out = pl.pallas_call(
        _pool_body, grid=grid,
        in_specs=[xspec, stspec, pspec, w(), w(), w(), w()],
        out_specs=pl.BlockSpec((_BLK, _OUTD), lambda i: (i, 0)),
        out_shape=jax.ShapeDtypeStruct((_B, _OUTD), f32),
        interpret=_INTERPRET,
    )(zc, stc, presence_scores, row(fine_g[2]), row(fine_beta[2]),
      out_w, row(out_b))
    return out


# speedup vs baseline: 1.0735x; 1.0735x over previous
import jax
import jax.numpy as jnp
from jax.experimental import pallas as pl

_B, _N, _F = 2048, 64, 64
_S, _SEGF, _OUTD, _H, _K = 4, 128, 128, 4, 4
_DH = _F // _H
_BLK = 128
_R = _BLK * _N
_STEPS = _B // _BLK
_TOT = float(_B * _N)

_INTERPRET = False


def _batched_dot(a, b, ca, cb):
    return jax.lax.dot_general(a, b, (((ca,), (cb,)), ((0,), (0,))))


def _compute_assign_t(x, sq, wq, bq, wk, bk):
    q = jax.lax.dot(sq, wq) + bq
    kk = jax.lax.dot(x, wk) + bk
    qt = jnp.transpose(q)
    qtile = jnp.concatenate([qt] * _H, axis=1)
    rows = jax.lax.broadcasted_iota(jnp.int32, (_F, _H * _S), 0) // _DH
    cols = jax.lax.broadcasted_iota(jnp.int32, (_F, _H * _S), 1) // _S
    qblk = jnp.where(rows == cols, qtile, 0.0)
    sc = jax.lax.dot(kk, qblk) * (1.0 / (_DH ** 0.5))
    sc3 = jnp.transpose(sc.reshape(_BLK, _N, _H * _S), (0, 2, 1))
    m = jnp.max(sc3, axis=2, keepdims=True)
    e = jnp.exp(sc3 - m)
    p = e / jnp.sum(e, axis=2, keepdims=True)
    attn = (p[:, 0 * _S:1 * _S] + p[:, 1 * _S:2 * _S] +
            p[:, 2 * _S:3 * _S] + p[:, 3 * _S:4 * _S]) * (1.0 / _H)
    m2 = jnp.max(attn, axis=1, keepdims=True)
    e2 = jnp.exp(attn - m2)
    return e2 / jnp.sum(e2, axis=1, keepdims=True)


def _knn_adj(px, py):
    dx = px[:, :, None] - px[:, None, :]
    dy = py[:, :, None] - py[:, None, :]
    d = jnp.sqrt(dx * dx + dy * dy + 1e-12)
    flane = jax.lax.broadcasted_iota(
        jnp.int32, (_BLK, _N, _N), 2).astype(jnp.float32)
    dcur = d
    sel0 = None
    for t in range(_K + 1):
        m = jnp.min(dcur, axis=2, keepdims=True)
        idxb = jnp.min(jnp.where(dcur == m, flane, float(_N)),
                       axis=2, keepdims=True)
        sel = flane == idxb
        if t == 0:
            sel0 = sel
        dcur = jnp.where(sel, jnp.inf, dcur)
    adj = jnp.where(sel0, 0.0,
                    jnp.where(dcur == jnp.inf, 1.0, 0.0))
    ones3 = jnp.ones((_BLK, 1, _N), jnp.float32)
    deg = _batched_dot(ones3, adj, 2, 1).reshape(_BLK, _N)
    deg = jnp.maximum(deg, 1.0)
    return adj, 1.0 / deg


def _message_from_adj(out, adj3, rdeg):
    o3 = out.reshape(_BLK, _N, _F)
    msgs = _batched_dot(adj3, o3, 1, 1)
    z3 = o3 + msgs * rdeg[:, :, None]
    return z3.reshape(_R, _F)


def _accum_stats(st_ref, z):
    @pl.when(pl.program_id(0) == 0)
    def _():
        st_ref[...] = jnp.zeros((8, 128), jnp.float32)
    ones = jnp.ones((1, _R), jnp.float32)
    s = jax.lax.dot(ones, z)
    q = jax.lax.dot(ones, z * z)
    upd = jnp.concatenate([s, q], axis=1)
    st_ref[0:1, :] = st_ref[0:1, :] + upd


def _norm_from_stats(st_ref, z, g, beta):
    st = st_ref[0:1, :]
    mu = st[:, 0:_F] * (1.0 / _TOT)
    var = st[:, _F:128] * (1.0 / _TOT) - mu * mu
    rstd = jax.lax.rsqrt(var + 1e-5)
    a = g * rstd
    b = beta - mu * a
    return jnp.maximum(a * z + b, 0.0)


def _seg_body(x_ref, px_ref, py_ref, sq_ref, wq_ref, bq_ref, wk_ref,
              bk_ref, a2sw_ref, a2sb_ref, seg_ref, at_ref, adj_ref,
              rdeg_ref):
    x = x_ref[...]
    at = _compute_assign_t(x, sq_ref[...], wq_ref[...], bq_ref[...],
                           wk_ref[...], bk_ref[...])
    at_ref[...] = at.reshape(_BLK * _S, _N)
    proj = jax.lax.dot(x, a2sw_ref[...]) + a2sb_ref[...]
    proj3 = proj.reshape(_BLK, _N, _SEGF)
    seg = _batched_dot(at, proj3, 2, 1)
    seg_ref[...] = seg.reshape(_BLK * _S, _SEGF)
    adj, rdeg = _knn_adj(px_ref[...], py_ref[...])
    adj_ref[...] = adj.reshape(_R, _N)
    rdeg_ref[...] = rdeg


def _coarse_body(sf_ref, w0, b0, g0, be0, w1, b1, g1, be1, out_ref):
    sf = sf_ref[...]
    for (w, b, g, be) in ((w0, b0, g0, be0), (w1, b1, g1, be1)):
        out = jax.lax.dot(sf, w[...]) + b[...]
        o3 = out.reshape(_B, _S, _SEGF)
        tot = jnp.sum(o3, axis=1, keepdims=True)
        z = (o3 + (tot - o3) * (1.0 / (_S - 1))).reshape(_B * _S, _SEGF)
        mu = jnp.mean(z, axis=0, keepdims=True)
        var = jnp.mean((z - mu) ** 2, axis=0, keepdims=True)
        sf = jnp.maximum(
            g[...] * (z - mu) * jax.lax.rsqrt(var + 1e-5) + be[...], 0.0)
    out_ref[...] = sf


def _fine0_body(x_ref, adj_ref, rdeg_ref, at_ref, seg2_ref, s2aw_ref,
                s2ab_ref, w_ref, b_ref, z_ref, st_ref):
    x = x_ref[...]
    at = at_ref[...].reshape(_BLK, _S, _N)
    seg2 = seg2_ref[...].reshape(_BLK, _S, _SEGF)
    segctx = _batched_dot(at, seg2, 1, 1)
    segctx = segctx.reshape(_R, _SEGF)
    enh = x + jax.lax.dot(segctx, s2aw_ref[...]) + s2ab_ref[...]
    out = jax.lax.dot(enh, w_ref[...]) + b_ref[...]
    z = _message_from_adj(out, adj_ref[...].reshape(_BLK, _N, _N),
                          rdeg_ref[...])
    z_ref[...] = z
    _accum_stats(st_ref, z)


def _fine_body(z_in_ref, stin_ref, adj_ref, rdeg_ref,
               g_ref, be_ref, w_ref, b_ref, z_ref, st_ref):
    y = _norm_from_stats(stin_ref, z_in_ref[...], g_ref[...], be_ref[...])
    out = jax.lax.dot(y, w_ref[...]) + b_ref[...]
    z = _message_from_adj(out, adj_ref[...].reshape(_BLK, _N, _N),
                          rdeg_ref[...])
    z_ref[...] = z
    _accum_stats(st_ref, z)


def _pool_body(z_in_ref, stin_ref, pres_ref, g_ref, be_ref,
               ow_ref, ob_ref, out_ref):
    y = _norm_from_stats(stin_ref, z_in_ref[...], g_ref[...], be_ref[...])
    y3 = y.reshape(_BLK, _N, _F)
    pw = pres_ref[...]
    num = _batched_dot(pw.reshape(_BLK, 1, _N), y3, 2, 1)
    num = num.reshape(_BLK, _F)
    den = jnp.maximum(jnp.sum(pw, axis=1, keepdims=True), 1e-8)
    gf = num / den
    out_ref[...] = jax.lax.dot(gf, ow_ref[...]) + ob_ref[...]


@jax.jit
def kernel(axle_features, axle_positions, presence_scores, seg_queries,
           mha_wq, mha_bq, mha_wk, mha_bk, a2s_w, a2s_b, s2a_w, s2a_b,
           coarse_w, coarse_b, coarse_g, coarse_beta,
           fine_w, fine_b, fine_g, fine_beta, out_w, out_b):
    f32 = jnp.float32
    x2 = axle_features.reshape(_B * _N, _F)
    row = lambda a: a.reshape(1, -1)

    grid = (_STEPS,)
    xspec = pl.BlockSpec((_R, _F), lambda i: (i, 0))
    pspec = pl.BlockSpec((_BLK, _N), lambda i: (i, 0))
    segspec = pl.BlockSpec((_BLK * _S, _SEGF), lambda i: (i, 0))
    atspec = pl.BlockSpec((_BLK * _S, _N), lambda i: (i, 0))
    stspec = pl.BlockSpec((8, 128), lambda i: (0, 0))
    w = lambda: pl.BlockSpec(None, lambda i: (0, 0))

    px = axle_positions[:, :, 0]
    py = axle_positions[:, :, 1]
    seg, at, adj, rdeg = pl.pallas_call(
        _seg_body, grid=grid,
        in_specs=[xspec, pspec, pspec] + [w()] * 7,
        out_specs=[segspec, atspec, xspec, pspec],
        out_shape=[jax.ShapeDtypeStruct((_B * _S, _SEGF), f32),
                   jax.ShapeDtypeStruct((_B * _S, _N), f32),
                   jax.ShapeDtypeStruct((_B * _N, _N), f32),
                   jax.ShapeDtypeStruct((_B, _N), f32)],
        interpret=_INTERPRET,
    )(x2, px, py, seg_queries, mha_wq, row(mha_bq), mha_wk,
      row(mha_bk), a2s_w, row(a2s_b))

    seg2 = pl.pallas_call(
        _coarse_body,
        in_specs=[pl.BlockSpec((_B * _S, _SEGF), lambda: (0, 0))] +
                 [pl.BlockSpec(None, lambda: (0, 0))] * 8,
        out_specs=pl.BlockSpec((_B * _S, _SEGF), lambda: (0, 0)),
        out_shape=jax.ShapeDtypeStruct((_B * _S, _SEGF), f32),
        interpret=_INTERPRET,
    )(seg, coarse_w[0], row(coarse_b[0]), row(coarse_g[0]),
      row(coarse_beta[0]), coarse_w[1], row(coarse_b[1]),
      row(coarse_g[1]), row(coarse_beta[1]))

    zshape = jax.ShapeDtypeStruct((_B * _N, _F), f32)
    stshape = jax.ShapeDtypeStruct((8, 128), f32)

    z1, st1 = pl.pallas_call(
        _fine0_body, grid=grid,
        in_specs=[xspec, xspec, pspec, atspec, segspec, w(), w(), w(), w()],
        out_specs=[xspec, stspec],
        out_shape=[zshape, stshape],
        interpret=_INTERPRET,
    )(x2, adj, rdeg, at, seg2, s2a_w, row(s2a_b), fine_w[0],
      row(fine_b[0]))

    zc, stc = z1, st1
    for i in (1, 2):
        zc, stc = pl.pallas_call(
            _fine_body, grid=grid,
            in_specs=[xspec, stspec, xspec, pspec, w(), w(), w(), w()],
            out_specs=[xspec, stspec],
            out_shape=[zshape, stshape],
            interpret=_INTERPRET,
        )(zc, stc, adj, rdeg, row(fine_g[i - 1]), row(fine_beta[i - 1]),
          fine_w[i], row(fine_b[i]))

    out = pl.pallas_call(
        _pool_body, grid=grid,
        in_specs=[xspec, stspec, pspec, w(), w(), w(), w()],
        out_specs=pl.BlockSpec((_BLK, _OUTD), lambda i: (i, 0)),
        out_shape=jax.ShapeDtypeStruct((_B, _OUTD), f32),
        interpret=_INTERPRET,
    )(zc, stc, presence_scores, row(fine_g[2]), row(fine_beta[2]),
      out_w, row(out_b))
    return out
